# baseline (speedup 1.0000x reference)
import jax
import jax.numpy as jnp
from jax.experimental import pallas as pl
from jax.experimental.pallas import tpu as pltpu

OBS_DIM = 55
A1_DIM = 10
A2_DIM = 10
IN_DIM = OBS_DIM + A1_DIM + A2_DIM
HIDDEN = 128
OUT_DIM = 35
NEG_SLOPE = 0.01

R = 8
_TILE_M = 256


def _mlp_kernel(obs_ref, a1_ref, a2_ref,
                w1o_ref, w1a1_ref, w1a2_ref, b1_ref,
                w2_ref, b2_ref,
                w3_ref, b3_ref,
                o_ref):
    f32 = jnp.float32
    bf16 = jnp.bfloat16
    h = (jnp.dot(obs_ref[...].astype(bf16), w1o_ref[...],
                 preferred_element_type=f32)
         + jnp.dot(a1_ref[...].astype(bf16), w1a1_ref[...],
                   preferred_element_type=f32)
         + jnp.dot(a2_ref[...].astype(bf16), w1a2_ref[...],
                   preferred_element_type=f32)
         + b1_ref[...])

    h = jnp.dot(h.astype(bf16), w2_ref[...],
                preferred_element_type=f32) + b2_ref[...]
    h = jnp.where(h >= 0, h, NEG_SLOPE * h)

    o_ref[...] = (jnp.dot(h.astype(bf16), w3_ref[...],
                          preferred_element_type=f32)
                  + b3_ref[...]).astype(o_ref.dtype)


def _block_diag(w, r):
    k, n = w.shape
    eye = jnp.eye(r, dtype=w.dtype)
    return jnp.kron(eye, w)


def kernel(observation, action_j1, action_j2, w1o, w1a, b1, w2, b2, w3, b3):
    B = observation.shape[0]
    bf16 = jnp.bfloat16
    f32 = jnp.float32

    rows = pl.cdiv(B, R)
    n_steps = max(2, pl.cdiv(rows, _TILE_M))
    tile_m = ((pl.cdiv(rows, n_steps) + 7) // 8) * 8
    rows_p = n_steps * tile_m
    Bp = rows_p * R
    if Bp != B:
        padr = Bp - B
        observation = jnp.pad(observation, ((0, padr), (0, 0)))
        action_j1 = jnp.pad(action_j1, ((0, padr), (0, 0)))
        action_j2 = jnp.pad(action_j2, ((0, padr), (0, 0)))

    obs_m = observation.reshape(rows_p, R * OBS_DIM)
    a1_m = action_j1.reshape(rows_p, R * A1_DIM)
    a2_m = action_j2.reshape(rows_p, R * A2_DIM)

    w1o_c = _block_diag(w1o.astype(bf16), R)
    w1a1_c = _block_diag(w1a[:A1_DIM, :].astype(bf16), R)
    w1a2_c = _block_diag(w1a[A1_DIM:, :].astype(bf16), R)
    w2_c = _block_diag(w2.astype(bf16), R)
    w3_c = _block_diag(w3.astype(bf16), R)
    b1_c = jnp.tile(b1.astype(f32), (1, R))
    b2_c = jnp.tile(b2.astype(f32), (1, R))
    b3_c = jnp.tile(b3.astype(f32), (1, R))

    def batch_spec(feat):
        return pl.BlockSpec((tile_m, feat), lambda i: (i, 0))

    def resident_spec(arr):
        return pl.BlockSpec(arr.shape, lambda i: (0, 0))

    weight_bytes = (2 * (w1o_c.size + w1a1_c.size + w1a2_c.size
                         + w2_c.size + w3_c.size)
                    + 4 * (b1_c.size + b2_c.size + b3_c.size))
    cost = pl.CostEstimate(
        flops=2 * rows_p * R * R * (IN_DIM * HIDDEN + HIDDEN * HIDDEN
                                    + HIDDEN * OUT_DIM),
        transcendentals=0,
        bytes_accessed=Bp * 4 * (IN_DIM + OUT_DIM) + weight_bytes)

    out_m = pl.pallas_call(
        _mlp_kernel,
        out_shape=jax.ShapeDtypeStruct((rows_p, R * OUT_DIM), f32),
        grid=(n_steps,),
        in_specs=[
            batch_spec(R * OBS_DIM), batch_spec(R * A1_DIM),
            batch_spec(R * A2_DIM),
            resident_spec(w1o_c), resident_spec(w1a1_c),
            resident_spec(w1a2_c), resident_spec(b1_c),
            resident_spec(w2_c), resident_spec(b2_c),
            resident_spec(w3_c), resident_spec(b3_c),
        ],
        out_specs=batch_spec(R * OUT_DIM),
        compiler_params=pltpu.CompilerParams(
            dimension_semantics=("parallel",)),
        cost_estimate=cost,
    )(obs_m, a1_m, a2_m,
      w1o_c, w1a1_c, w1a2_c, b1_c, w2_c, b2_c, w3_c, b3_c)

    out = out_m.reshape(Bp, OUT_DIM)
    return out[:B] if Bp != B else out


# speedup vs baseline: 1.7096x; 1.7096x over previous
import jax
import jax.numpy as jnp
from jax.experimental import pallas as pl
from jax.experimental.pallas import tpu as pltpu

OBS_DIM = 55
A1_DIM = 10
A2_DIM = 10
IN_DIM = OBS_DIM + A1_DIM + A2_DIM
HIDDEN = 128
OUT_DIM = 35
NEG_SLOPE = 0.01

_TILE_B = 8192
_SINGLE_STEP_MAX_B = 511


def _mlp_kernel(obs_ref, a1_ref, a2_ref,
                w1o_ref, w1a1_ref, w1a2_ref, b1_ref,
                w2_ref, b2_ref,
                w3_ref, b3_ref,
                o_ref):
    f32 = jnp.float32
    bf16 = jnp.bfloat16
    h = (jnp.dot(obs_ref[...].astype(bf16), w1o_ref[...],
                 preferred_element_type=f32)
         + jnp.dot(a1_ref[...].astype(bf16), w1a1_ref[...],
                   preferred_element_type=f32)
         + jnp.dot(a2_ref[...].astype(bf16), w1a2_ref[...],
                   preferred_element_type=f32)
         + b1_ref[...])

    h = jnp.dot(h.astype(bf16), w2_ref[...],
                preferred_element_type=f32) + b2_ref[...]
    h = jnp.where(h >= 0, h, NEG_SLOPE * h)

    o_ref[...] = (jnp.dot(h.astype(bf16), w3_ref[...],
                          preferred_element_type=f32)
                  + b3_ref[...]).astype(o_ref.dtype)


def _choose_tiling(B):
    if B <= _SINGLE_STEP_MAX_B:
        return 1, B
    n_steps = max(2, pl.cdiv(B, _TILE_B))
    tile_b = pl.cdiv(B, n_steps)
    tile_b = ((tile_b + 7) // 8) * 8
    return n_steps, tile_b


def kernel(observation, action_j1, action_j2, w1o, w1a, b1, w2, b2, w3, b3):
    B = observation.shape[0]

    bf16 = jnp.bfloat16
    w1o_c = w1o.astype(bf16)
    w1a1_c = w1a[:A1_DIM, :].astype(bf16)
    w1a2_c = w1a[A1_DIM:, :].astype(bf16)
    w2_c = w2.astype(bf16)
    w3_c = w3.astype(bf16)
    b1_c = b1.astype(jnp.float32)
    b2_c = b2.astype(jnp.float32)
    b3_c = b3.astype(jnp.float32)

    n_steps, tile_b = _choose_tiling(B)
    Bp = n_steps * tile_b
    pad = Bp - B
    if pad:
        observation = jnp.pad(observation, ((0, pad), (0, 0)))
        action_j1 = jnp.pad(action_j1, ((0, pad), (0, 0)))
        action_j2 = jnp.pad(action_j2, ((0, pad), (0, 0)))

    def batch_spec(feat):
        return pl.BlockSpec((tile_b, feat), lambda i: (i, 0))

    def resident_spec(arr):
        return pl.BlockSpec(arr.shape, lambda i: (0, 0))

    weight_bytes = (2 * (w1o_c.size + w1a1_c.size + w1a2_c.size
                         + w2_c.size + w3_c.size)
                    + 4 * (b1_c.size + b2_c.size + b3_c.size))
    cost = pl.CostEstimate(
        flops=2 * Bp * (IN_DIM * HIDDEN + HIDDEN * HIDDEN + HIDDEN * OUT_DIM),
        transcendentals=0,
        bytes_accessed=Bp * 4 * (IN_DIM + OUT_DIM) + weight_bytes)

    out = pl.pallas_call(
        _mlp_kernel,
        out_shape=jax.ShapeDtypeStruct((Bp, OUT_DIM), jnp.float32),
        grid=(n_steps,),
        in_specs=[
            batch_spec(OBS_DIM), batch_spec(A1_DIM), batch_spec(A2_DIM),
            resident_spec(w1o_c), resident_spec(w1a1_c),
            resident_spec(w1a2_c), resident_spec(b1_c),
            resident_spec(w2_c), resident_spec(b2_c),
            resident_spec(w3_c), resident_spec(b3_c),
        ],
        out_specs=batch_spec(OUT_DIM),
        compiler_params=pltpu.CompilerParams(
            dimension_semantics=("parallel",)),
        cost_estimate=cost,
    )(observation, action_j1, action_j2,
      w1o_c, w1a1_c, w1a2_c, b1_c, w2_c, b2_c, w3_c, b3_c)

    return out[:B] if pad else out


# speedup vs baseline: 1.9405x; 1.1350x over previous
import jax
import jax.numpy as jnp
from jax.experimental import pallas as pl
from jax.experimental.pallas import tpu as pltpu

OBS_DIM = 55
A1_DIM = 10
A2_DIM = 10
IN_DIM = OBS_DIM + A1_DIM + A2_DIM
HIDDEN = 128
OUT_DIM = 35
NEG_SLOPE = 0.01

_TILE_B = 8192
_SINGLE_STEP_MAX_B = 511


def _mlp_kernel(obs_ref, a1_ref, a2_ref,
                w1o_ref, w1a1_ref, w1a2_ref, b1_ref,
                w2_ref, b2_ref,
                w3_ref, b3_ref,
                o_ref):
    f32 = jnp.float32
    bf16 = jnp.bfloat16
    h = (jnp.dot(obs_ref[...], w1o_ref[...],
                 preferred_element_type=f32)
         + jnp.dot(a1_ref[...], w1a1_ref[...],
                   preferred_element_type=f32)
         + jnp.dot(a2_ref[...], w1a2_ref[...],
                   preferred_element_type=f32)
         + b1_ref[...])

    h = jnp.dot(h.astype(bf16), w2_ref[...],
                preferred_element_type=f32) + b2_ref[...]
    h = jnp.where(h >= 0, h, NEG_SLOPE * h)

    o_ref[...] = (jnp.dot(h.astype(bf16), w3_ref[...],
                          preferred_element_type=f32)
                  + b3_ref[...]).astype(o_ref.dtype)


def _choose_tiling(B):
    if B <= _SINGLE_STEP_MAX_B:
        return 1, B
    n_steps = max(2, pl.cdiv(B, _TILE_B))
    tile_b = pl.cdiv(B, n_steps)
    tile_b = ((tile_b + 7) // 8) * 8
    return n_steps, tile_b


def kernel(observation, action_j1, action_j2, w1o, w1a, b1, w2, b2, w3, b3):
    B = observation.shape[0]

    bf16 = jnp.bfloat16
    w1o_c = w1o.astype(bf16)
    w1a1_c = w1a[:A1_DIM, :].astype(bf16)
    w1a2_c = w1a[A1_DIM:, :].astype(bf16)
    w2_c = w2.astype(bf16)
    w3_c = w3.astype(bf16)
    b1_c = b1.astype(jnp.float32)
    b2_c = b2.astype(jnp.float32)
    b3_c = b3.astype(jnp.float32)

    observation = observation.astype(bf16)
    action_j1 = action_j1.astype(bf16)
    action_j2 = action_j2.astype(bf16)

    n_steps, tile_b = _choose_tiling(B)
    Bp = n_steps * tile_b
    pad = Bp - B
    if pad:
        observation = jnp.pad(observation, ((0, pad), (0, 0)))
        action_j1 = jnp.pad(action_j1, ((0, pad), (0, 0)))
        action_j2 = jnp.pad(action_j2, ((0, pad), (0, 0)))

    def batch_spec(feat):
        return pl.BlockSpec((tile_b, feat), lambda i: (i, 0))

    def resident_spec(arr):
        return pl.BlockSpec(arr.shape, lambda i: (0, 0))

    weight_bytes = (2 * (w1o_c.size + w1a1_c.size + w1a2_c.size
                         + w2_c.size + w3_c.size)
                    + 4 * (b1_c.size + b2_c.size + b3_c.size))
    cost = pl.CostEstimate(
        flops=2 * Bp * (IN_DIM * HIDDEN + HIDDEN * HIDDEN + HIDDEN * OUT_DIM),
        transcendentals=0,
        bytes_accessed=Bp * 4 * (IN_DIM + OUT_DIM) + weight_bytes)

    out = pl.pallas_call(
        _mlp_kernel,
        out_shape=jax.ShapeDtypeStruct((Bp, OUT_DIM), jnp.float32),
        grid=(n_steps,),
        in_specs=[
            batch_spec(OBS_DIM), batch_spec(A1_DIM), batch_spec(A2_DIM),
            resident_spec(w1o_c), resident_spec(w1a1_c),
            resident_spec(w1a2_c), resident_spec(b1_c),
            resident_spec(w2_c), resident_spec(b2_c),
            resident_spec(w3_c), resident_spec(b3_c),
        ],
        out_specs=batch_spec(OUT_DIM),
        compiler_params=pltpu.CompilerParams(
            dimension_semantics=("parallel",)),
        cost_estimate=cost,
    )(observation, action_j1, action_j2,
      w1o_c, w1a1_c, w1a2_c, b1_c, w2_c, b2_c, w3_c, b3_c)

    return out[:B] if pad else out


# speedup vs baseline: 2.0048x; 1.0331x over previous
import jax
import jax.numpy as jnp
from jax.experimental import pallas as pl
from jax.experimental.pallas import tpu as pltpu

OBS_DIM = 55
A1_DIM = 10
A2_DIM = 10
IN_DIM = OBS_DIM + A1_DIM + A2_DIM
HIDDEN = 128
OUT_DIM = 35
NEG_SLOPE = 0.01

_TILE_B = 8192
_SINGLE_STEP_MAX_B = 511


def _mlp_kernel(x_ref,
                w1_ref, b1_ref,
                w2_ref, b2_ref,
                w3_ref, b3_ref,
                o_ref):
    f32 = jnp.float32
    bf16 = jnp.bfloat16
    h = (jnp.dot(x_ref[...], w1_ref[...], preferred_element_type=f32)
         + b1_ref[...])

    h = jnp.dot(h.astype(bf16), w2_ref[...],
                preferred_element_type=f32) + b2_ref[...]
    h = jnp.where(h >= 0, h, NEG_SLOPE * h)

    o_ref[...] = (jnp.dot(h.astype(bf16), w3_ref[...],
                          preferred_element_type=f32)
                  + b3_ref[...]).astype(o_ref.dtype)


def _choose_tiling(B):
    if B <= _SINGLE_STEP_MAX_B:
        return 1, B
    n_steps = max(2, pl.cdiv(B, _TILE_B))
    tile_b = pl.cdiv(B, n_steps)
    tile_b = ((tile_b + 7) // 8) * 8
    return n_steps, tile_b


def kernel(observation, action_j1, action_j2, w1o, w1a, b1, w2, b2, w3, b3):
    B = observation.shape[0]
    bf16 = jnp.bfloat16
    f32 = jnp.float32

    w1_c = jnp.concatenate([w1o, w1a], axis=0).astype(bf16)
    w2_c = w2.astype(bf16)
    w3_c = w3.astype(bf16)
    b1_c = b1.astype(f32)
    b2_c = b2.astype(f32)
    b3_c = b3.astype(f32)

    x = jnp.concatenate([observation, action_j1, action_j2],
                        axis=1).astype(bf16)

    n_steps, tile_b = _choose_tiling(B)
    Bp = n_steps * tile_b
    pad = Bp - B
    if pad:
        x = jnp.pad(x, ((0, pad), (0, 0)))

    def batch_spec(feat):
        return pl.BlockSpec((tile_b, feat), lambda i: (i, 0))

    def resident_spec(arr):
        return pl.BlockSpec(arr.shape, lambda i: (0, 0))

    weight_bytes = (2 * (w1_c.size + w2_c.size + w3_c.size)
                    + 4 * (b1_c.size + b2_c.size + b3_c.size))
    cost = pl.CostEstimate(
        flops=2 * Bp * (IN_DIM * HIDDEN + HIDDEN * HIDDEN + HIDDEN * OUT_DIM),
        transcendentals=0,
        bytes_accessed=Bp * (2 * IN_DIM + 4 * OUT_DIM) + weight_bytes)

    out = pl.pallas_call(
        _mlp_kernel,
        out_shape=jax.ShapeDtypeStruct((Bp, OUT_DIM), f32),
        grid=(n_steps,),
        in_specs=[
            batch_spec(IN_DIM),
            resident_spec(w1_c), resident_spec(b1_c),
            resident_spec(w2_c), resident_spec(b2_c),
            resident_spec(w3_c), resident_spec(b3_c),
        ],
        out_specs=batch_spec(OUT_DIM),
        compiler_params=pltpu.CompilerParams(
            dimension_semantics=("parallel",)),
        cost_estimate=cost,
    )(x, w1_c, b1_c, w2_c, b2_c, w3_c, b3_c)

    return out[:B] if pad else out


# speedup vs baseline: 2.0480x; 1.0216x over previous
import jax
import jax.numpy as jnp
from jax.experimental import pallas as pl
from jax.experimental.pallas import tpu as pltpu

OBS_DIM = 55
A1_DIM = 10
A2_DIM = 10
IN_DIM = OBS_DIM + A1_DIM + A2_DIM
HIDDEN = 128
OUT_DIM = 35
NEG_SLOPE = 0.01

_TILE_B = 16384
_SINGLE_STEP_MAX_B = 511


def _mlp_kernel(x_ref,
                w1_ref, b1_ref,
                w2_ref, b2_ref,
                w3_ref, b3_ref,
                o_ref):
    f32 = jnp.float32
    bf16 = jnp.bfloat16
    h = (jnp.dot(x_ref[...], w1_ref[...], preferred_element_type=f32)
         + b1_ref[...])

    h = jnp.dot(h.astype(bf16), w2_ref[...],
                preferred_element_type=f32) + b2_ref[...]
    h = jnp.where(h >= 0, h, NEG_SLOPE * h)

    o_ref[...] = (jnp.dot(h.astype(bf16), w3_ref[...],
                          preferred_element_type=f32)
                  + b3_ref[...]).astype(o_ref.dtype)


def _choose_tiling(B):
    if B <= _SINGLE_STEP_MAX_B:
        return 1, B
    n_steps = max(2, pl.cdiv(B, _TILE_B))
    tile_b = pl.cdiv(B, n_steps)
    tile_b = ((tile_b + 7) // 8) * 8
    return n_steps, tile_b


def kernel(observation, action_j1, action_j2, w1o, w1a, b1, w2, b2, w3, b3):
    B = observation.shape[0]
    bf16 = jnp.bfloat16
    f32 = jnp.float32

    w1_c = jnp.concatenate([w1o, w1a], axis=0).astype(bf16)
    w2_c = w2.astype(bf16)
    w3_c = w3.astype(bf16)
    b1_c = b1.astype(f32)
    b2_c = b2.astype(f32)
    b3_c = b3.astype(f32)

    x = jnp.concatenate([observation, action_j1, action_j2],
                        axis=1).astype(bf16)

    n_steps, tile_b = _choose_tiling(B)
    Bp = n_steps * tile_b
    pad = Bp - B
    if pad:
        x = jnp.pad(x, ((0, pad), (0, 0)))

    def batch_spec(feat):
        return pl.BlockSpec((tile_b, feat), lambda i: (i, 0))

    def resident_spec(arr):
        return pl.BlockSpec(arr.shape, lambda i: (0, 0))

    weight_bytes = (2 * (w1_c.size + w2_c.size + w3_c.size)
                    + 4 * (b1_c.size + b2_c.size + b3_c.size))
    cost = pl.CostEstimate(
        flops=2 * Bp * (IN_DIM * HIDDEN + HIDDEN * HIDDEN + HIDDEN * OUT_DIM),
        transcendentals=0,
        bytes_accessed=Bp * (2 * IN_DIM + 4 * OUT_DIM) + weight_bytes)

    out = pl.pallas_call(
        _mlp_kernel,
        out_shape=jax.ShapeDtypeStruct((Bp, OUT_DIM), f32),
        grid=(n_steps,),
        in_specs=[
            batch_spec(IN_DIM),
            resident_spec(w1_c), resident_spec(b1_c),
            resident_spec(w2_c), resident_spec(b2_c),
            resident_spec(w3_c), resident_spec(b3_c),
        ],
        out_specs=batch_spec(OUT_DIM),
        compiler_params=pltpu.CompilerParams(
            dimension_semantics=("parallel",)),
        cost_estimate=cost,
    )(x, w1_c, b1_c, w2_c, b2_c, w3_c, b3_c)

    return out[:B] if pad else out
